# baseline (speedup 1.0000x reference)
import jax
import jax.numpy as jnp
from jax.experimental import pallas as pl
from jax.experimental.pallas import tpu as pltpu
from jax.experimental.pallas import tpu_sc as plsc

N = 10000
NPAD = 10240
CIN = 128
COUT = 128
MID = 128
S = 8
K = 16
EPS = 1e-5

QB = 512
QB2 = 256
GW = 128

_INF = float("inf")


NB = 40
BW = 256
R = 4


def _knn_body(a_ref, bt_ref, x2c_ref, out_ref):
    a = a_ref[...]
    bt = bt_ref[...]
    d2 = jax.lax.dot_general(a, bt, (((1,), (0,)), ((), ())),
                             preferred_element_type=jnp.float32)
    x2q = 0.25 * jnp.sum(a * a, axis=1, keepdims=True)
    d = (d2 + x2c_ref[...]) + x2q
    p_iota = jax.lax.broadcasted_iota(jnp.int32, (QB, BW), 1)

    def sl(t):
        s = d[:, t * BW:(t + 1) * BW]
        if (t + 1) * BW > N:
            s = jnp.where(p_iota >= N - t * BW, _INF, s)
        return s

    Ms, As = [], []
    for r in range(R):
        Mr = jnp.full((QB, BW), _INF, jnp.float32)
        Ar = jnp.zeros((QB, BW), jnp.int32)
        for t in range(NB):
            dt = sl(t)
            for Ap in As:
                dt = jnp.where(Ap == t, _INF, dt)
            c = dt < Mr
            Mr = jnp.where(c, dt, Mr)
            Ar = jnp.where(c, t, Ar)
        Ms.append(Mr)
        As.append(Ar)
    MM = jnp.concatenate(Ms, axis=1)
    II = jnp.concatenate([Ar * BW + p_iota for Ar in As], axis=1)
    cols = []
    for _ in range(K):
        v = jnp.min(MM, axis=1, keepdims=True)
        cand = jnp.where(MM == v, II, NPAD)
        ik = jnp.min(cand, axis=1, keepdims=True)
        cols.append(ik)
        MM = jnp.where(II == ik, _INF, MM)
    out_ref[...] = jnp.concatenate(cols, axis=1)


def _knn_topk(a_pad, bt_pad, x2c):
    return pl.pallas_call(
        _knn_body,
        grid=(a_pad.shape[0] // QB,),
        in_specs=[
            pl.BlockSpec((QB, 8), lambda i: (i, 0)),
            pl.BlockSpec((8, NPAD), lambda i: (0, 0)),
            pl.BlockSpec((1, NPAD), lambda i: (0, 0)),
        ],
        out_specs=pl.BlockSpec((QB, K), lambda i: (i, 0)),
        out_shape=jax.ShapeDtypeStruct((a_pad.shape[0], K), jnp.int32),
    )(a_pad, bt_pad, x2c)


def _sc_gather(kv_pad, pt_pad, idx_flat):
    nidx = idx_flat.shape[0]
    idx2 = idx_flat.reshape(1, nidx)
    mesh = plsc.VectorSubcoreMesh(core_axis_name="c", subcore_axis_name="s")

    @pl.kernel(
        out_type=(
            jax.ShapeDtypeStruct((nidx, 2 * CIN), jnp.float32),
            jax.ShapeDtypeStruct((nidx, 128), jnp.float32),
        ),
        mesh=mesh,
    )
    def gather_kernel(kv_hbm, pt_hbm, i_hbm, of_hbm, op_hbm):
        def body(i_vmem, of_vmem, op_vmem):
            pltpu.sync_copy(kv_hbm.at[i_vmem.at[0]], of_vmem)
            pltpu.sync_copy(pt_hbm.at[i_vmem.at[0]], op_vmem)

        pltpu.emit_pipeline(
            body,
            grid=(nidx // GW,),
            in_specs=[pl.BlockSpec((1, GW), lambda i: (0, i))],
            out_specs=[
                pl.BlockSpec((GW, 2 * CIN), lambda i: (i, 0)),
                pl.BlockSpec((GW, 128), lambda i: (i, 0)),
            ],
            core_axis_name=("c", "s"),
            dimension_semantics=(pltpu.PARALLEL,),
        )(i_hbm, of_hbm, op_hbm)

    return gather_kernel(kv_pad, pt_pad, idx2)


def _proj_body(feat_ref, Wq_ref, bq_ref, Wk_ref, bk_ref, Wv_ref, bv_ref,
               q_ref, kv_ref):
    f = feat_ref[...]
    q_ref[...] = _mm(f, Wq_ref[...]) + bq_ref[...]
    fk = _mm(f, Wk_ref[...]) + bk_ref[...]
    fv = _mm(f, Wv_ref[...]) + bv_ref[...]
    kv_ref[...] = jnp.concatenate([fk, fv], axis=1)


def _project(feat_pad, Wq, bq, Wk, bk, Wv, bv):
    full = lambda shape: pl.BlockSpec(shape, lambda i: tuple(0 for _ in shape))
    PB = 512
    return pl.pallas_call(
        _proj_body,
        grid=(NPAD // PB,),
        in_specs=[pl.BlockSpec((PB, CIN), lambda i: (i, 0)),
                  full(Wq.shape), full(bq.shape), full(Wk.shape),
                  full(bk.shape), full(Wv.shape), full(bv.shape)],
        out_specs=[pl.BlockSpec((PB, MID), lambda i: (i, 0)),
                   pl.BlockSpec((PB, 2 * CIN), lambda i: (i, 0))],
        out_shape=[jax.ShapeDtypeStruct((NPAD, MID), jnp.float32),
                   jax.ShapeDtypeStruct((NPAD, 2 * CIN), jnp.float32)],
    )(feat_pad, Wq, bq, Wk, bk, Wv, bv)


def _mm(a, b):
    return jax.lax.dot_general(a, b, (((1,), (0,)), ((), ())),
                               preferred_element_type=jnp.float32)


def _attn_body(gkv_ref, gpt_ref, fq_ref, pt_ref,
               Wp1_ref, bp1_ref, gp1_ref, bep1_ref, Wp2_ref, bp2_ref,
               gw1_ref, bew1_ref, Ww1_ref, bw1_ref,
               gw2_ref, bew2_ref, Ww2_ref, bw2_ref, bmat_ref, out_ref):
    inv = (1.0 + EPS) ** -0.5
    gkv = gkv_ref[...]
    fk = gkv[:, :CIN]
    fv = gkv[:, CIN:]
    fq = fq_ref[...]
    rel = (gpt_ref[...][:, :16].reshape(QB2, K, 16)
           - pt_ref[...][:, :16][:, None, :])
    pr = _mm(rel.reshape(QB2 * K, 16), Wp1_ref[...]) + bp1_ref[...]
    pr = jnp.maximum(gp1_ref[...] * pr * inv + bep1_ref[...], 0.0)
    pr = _mm(pr, Wp2_ref[...]) + bp2_ref[...]
    w = fk.reshape(QB2, K, COUT) - fq[:, None, :] + pr.reshape(QB2, K, COUT)
    w = jnp.maximum(gw1_ref[...] * w * inv + bew1_ref[...], 0.0)
    w = _mm(w.reshape(QB2 * K, MID), Ww1_ref[...]) + bw1_ref[...]
    w = jnp.maximum(gw2_ref[...] * w * inv + bew2_ref[...], 0.0)
    w = _mm(w, Ww2_ref[...]) + bw2_ref[...]
    w3 = w.reshape(QB2, K, COUT // S)
    w3 = w3 - jnp.max(w3, axis=1, keepdims=True)
    e = jnp.exp(w3)
    sm = e / jnp.sum(e, axis=1, keepdims=True)
    w128 = _mm(sm.reshape(QB2 * K, COUT // S), bmat_ref[...])
    val = (fv + pr).reshape(QB2, K, COUT) * w128.reshape(QB2, K, COUT)
    out_ref[...] = jnp.sum(val, axis=1)


def _attention(gkv, gpt, fq_tab, pt_pad, params):
    full = lambda shape: pl.BlockSpec(shape, lambda i: tuple(0 for _ in shape))
    in_specs = [
        pl.BlockSpec((QB2 * K, 2 * CIN), lambda i: (i, 0)),
        pl.BlockSpec((QB2 * K, 128), lambda i: (i, 0)),
        pl.BlockSpec((QB2, CIN), lambda i: (i, 0)),
        pl.BlockSpec((QB2, 128), lambda i: (i, 0)),
    ] + [full(p.shape) for p in params]
    nrows = fq_tab.shape[0]
    return pl.pallas_call(
        _attn_body,
        grid=(nrows // QB2,),
        in_specs=in_specs,
        out_specs=pl.BlockSpec((QB2, COUT), lambda i: (i, 0)),
        out_shape=jax.ShapeDtypeStruct((nrows, COUT), jnp.float32),
    )(gkv, gpt, fq_tab, pt_pad, *params)


def kernel(point, feat, row_splits, training, Wq, bq, Wk, bk, Wv, bv,
           Wp1, bp1, gp1, bep1, Wp2, bp2, gw1, bew1, Ww1, bw1,
           gw2, bew2, Ww2, bw2):
    pad = NPAD - N
    x2c = jnp.pad(jnp.sum(point * point, axis=1)[None, :],
                  ((0, 0), (0, pad)))
    a_pad = jnp.pad(-2.0 * point, ((0, pad), (0, 5)))
    bt_pad = jnp.pad(point, ((0, pad), (0, 5))).T
    pt_pad = jnp.pad(point, ((0, pad), (0, 125)))
    feat_pad = jnp.pad(feat, ((0, pad), (0, 0)))

    row = lambda v: v.reshape(1, -1)
    pad16 = lambda v: jnp.pad(v, (0, 16 - v.shape[0])).reshape(1, 16)
    fq_tab, kv_tab = _project(feat_pad, Wq, row(bq), Wk, row(bk),
                              Wv, row(bv))

    nchunks = 4
    cw = NPAD // nchunks
    gkvs, gpts = [], []
    for h in range(nchunks):
        idx_h = _knn_topk(a_pad[h * cw:(h + 1) * cw], bt_pad, x2c)
        g1, g2 = _sc_gather(kv_tab, pt_pad, idx_h.reshape(-1))
        gkvs.append(g1)
        gpts.append(g2)

    Wp1p = jnp.zeros((16, 16), jnp.float32).at[:3, :3].set(Wp1)
    Wp2p = jnp.zeros((16, COUT), jnp.float32).at[:3, :].set(Wp2)
    bmat = (jax.lax.broadcasted_iota(jnp.int32, (16, COUT), 1) % 16
            == jax.lax.broadcasted_iota(jnp.int32, (16, COUT), 0)
            ).astype(jnp.float32)

    params = [Wp1p, pad16(bp1), pad16(gp1), pad16(bep1), Wp2p, row(bp2),
              row(gw1), row(bew1), Ww1, row(bw1),
              pad16(gw2), pad16(bew2), Ww2, pad16(bw2), bmat]
    outs = [_attention(gkvs[h], gpts[h], fq_tab[h * cw:(h + 1) * cw],
                       pt_pad[h * cw:(h + 1) * cw], params)
            for h in range(nchunks)]
    return jnp.concatenate(outs, axis=0)[:N]


# speedup vs baseline: 1.0746x; 1.0746x over previous
import jax
import jax.numpy as jnp
from jax.experimental import pallas as pl
from jax.experimental.pallas import tpu as pltpu
from jax.experimental.pallas import tpu_sc as plsc

N = 10000
NPAD = 10240
CIN = 128
COUT = 128
MID = 128
S = 8
K = 16
EPS = 1e-5

QB = 512
QB2 = 256
GW = 128

_INF = float("inf")


NB = 40
BW = 256
R = 4


def _knn_body(a_ref, bt_ref, x2c_ref, out_ref):
    a = a_ref[...]
    bt = bt_ref[...]
    d2 = jax.lax.dot_general(a, bt, (((1,), (0,)), ((), ())),
                             preferred_element_type=jnp.float32)
    x2q = 0.25 * jnp.sum(a * a, axis=1, keepdims=True)
    d = (d2 + x2c_ref[...]) + x2q
    p_iota = jax.lax.broadcasted_iota(jnp.int32, (QB, BW), 1)

    def sl(t):
        s = d[:, t * BW:(t + 1) * BW]
        if (t + 1) * BW > N:
            s = jnp.where(p_iota >= N - t * BW, _INF, s)
        return s

    Ms, As = [], []
    for r in range(R):
        Mr = jnp.full((QB, BW), _INF, jnp.float32)
        Ar = jnp.zeros((QB, BW), jnp.int32)
        for t in range(NB):
            dt = sl(t)
            for Ap in As:
                dt = jnp.where(Ap == t, _INF, dt)
            c = dt < Mr
            Mr = jnp.where(c, dt, Mr)
            Ar = jnp.where(c, t, Ar)
        Ms.append(Mr)
        As.append(Ar)
    MM = jnp.concatenate(Ms, axis=1)
    II = jnp.concatenate([Ar * BW + p_iota for Ar in As], axis=1)
    cols = []
    for _ in range(K):
        v = jnp.min(MM, axis=1, keepdims=True)
        cand = jnp.where(MM == v, II, NPAD)
        ik = jnp.min(cand, axis=1, keepdims=True)
        cols.append(ik)
        MM = jnp.where(II == ik, _INF, MM)
    out_ref[...] = jnp.concatenate(cols, axis=1)


def _knn_topk(a_pad, bt_pad, x2c):
    return pl.pallas_call(
        _knn_body,
        grid=(a_pad.shape[0] // QB,),
        in_specs=[
            pl.BlockSpec((QB, 8), lambda i: (i, 0)),
            pl.BlockSpec((8, NPAD), lambda i: (0, 0)),
            pl.BlockSpec((1, NPAD), lambda i: (0, 0)),
        ],
        out_specs=pl.BlockSpec((QB, K), lambda i: (i, 0)),
        out_shape=jax.ShapeDtypeStruct((a_pad.shape[0], K), jnp.int32),
    )(a_pad, bt_pad, x2c)


def _sc_gather(feat_pad, pt_pad, idx_flat):
    nidx = idx_flat.shape[0]
    idx2 = idx_flat.reshape(1, nidx)
    mesh = plsc.VectorSubcoreMesh(core_axis_name="c", subcore_axis_name="s")

    @pl.kernel(
        out_type=(
            jax.ShapeDtypeStruct((nidx, CIN), jnp.float32),
            jax.ShapeDtypeStruct((nidx, 128), jnp.float32),
        ),
        mesh=mesh,
    )
    def gather_kernel(feat_hbm, pt_hbm, i_hbm, of_hbm, op_hbm):
        def body(i_vmem, of_vmem, op_vmem):
            pltpu.sync_copy(feat_hbm.at[i_vmem.at[0]], of_vmem)
            pltpu.sync_copy(pt_hbm.at[i_vmem.at[0]], op_vmem)

        pltpu.emit_pipeline(
            body,
            grid=(nidx // GW,),
            in_specs=[pl.BlockSpec((1, GW), lambda i: (0, i))],
            out_specs=[
                pl.BlockSpec((GW, CIN), lambda i: (i, 0)),
                pl.BlockSpec((GW, 128), lambda i: (i, 0)),
            ],
            core_axis_name=("c", "s"),
            dimension_semantics=(pltpu.PARALLEL,),
        )(i_hbm, of_hbm, op_hbm)

    return gather_kernel(feat_pad, pt_pad, idx2)


def _mm(a, b):
    return jax.lax.dot_general(a, b, (((1,), (0,)), ((), ())),
                               preferred_element_type=jnp.float32)


def _attn_body(gfeat_ref, gpt_ref, feat_ref, pt_ref,
               Wq_ref, bq_ref, Wk_ref, bk_ref, Wv_ref, bv_ref,
               Wp1_ref, bp1_ref, gp1_ref, bep1_ref, Wp2_ref, bp2_ref,
               gw1_ref, bew1_ref, Ww1_ref, bw1_ref,
               gw2_ref, bew2_ref, Ww2_ref, bw2_ref, bmat_ref, out_ref):
    inv = (1.0 + EPS) ** -0.5
    gf = gfeat_ref[...]
    fk = _mm(gf, Wk_ref[...]) + bk_ref[...]
    fv = _mm(gf, Wv_ref[...]) + bv_ref[...]
    fq = _mm(feat_ref[...], Wq_ref[...]) + bq_ref[...]
    rel = (gpt_ref[...][:, :16].reshape(QB2, K, 16)
           - pt_ref[...][:, :16][:, None, :])
    pr = _mm(rel.reshape(QB2 * K, 16), Wp1_ref[...]) + bp1_ref[...]
    pr = jnp.maximum(gp1_ref[...] * pr * inv + bep1_ref[...], 0.0)
    pr = _mm(pr, Wp2_ref[...]) + bp2_ref[...]
    w = fk.reshape(QB2, K, COUT) - fq[:, None, :] + pr.reshape(QB2, K, COUT)
    w = jnp.maximum(gw1_ref[...] * w * inv + bew1_ref[...], 0.0)
    w = _mm(w.reshape(QB2 * K, MID), Ww1_ref[...]) + bw1_ref[...]
    w = jnp.maximum(gw2_ref[...] * w * inv + bew2_ref[...], 0.0)
    w = _mm(w, Ww2_ref[...]) + bw2_ref[...]
    w3 = w.reshape(QB2, K, COUT // S)
    w3 = w3 - jnp.max(w3, axis=1, keepdims=True)
    e = jnp.exp(w3)
    sm = e / jnp.sum(e, axis=1, keepdims=True)
    w128 = _mm(sm.reshape(QB2 * K, COUT // S), bmat_ref[...])
    val = (fv + pr).reshape(QB2, K, COUT) * w128.reshape(QB2, K, COUT)
    out_ref[...] = jnp.sum(val, axis=1)


def _attention(gfeat, gpt, feat_pad, pt_pad, params):
    full = lambda shape: pl.BlockSpec(shape, lambda i: tuple(0 for _ in shape))
    in_specs = [
        pl.BlockSpec((QB2 * K, CIN), lambda i: (i, 0)),
        pl.BlockSpec((QB2 * K, 128), lambda i: (i, 0)),
        pl.BlockSpec((QB2, CIN), lambda i: (i, 0)),
        pl.BlockSpec((QB2, 128), lambda i: (i, 0)),
    ] + [full(p.shape) for p in params]
    nrows = feat_pad.shape[0]
    return pl.pallas_call(
        _attn_body,
        grid=(nrows // QB2,),
        in_specs=in_specs,
        out_specs=pl.BlockSpec((QB2, COUT), lambda i: (i, 0)),
        out_shape=jax.ShapeDtypeStruct((nrows, COUT), jnp.float32),
    )(gfeat, gpt, feat_pad, pt_pad, *params)


def kernel(point, feat, row_splits, training, Wq, bq, Wk, bk, Wv, bv,
           Wp1, bp1, gp1, bep1, Wp2, bp2, gw1, bew1, Ww1, bw1,
           gw2, bew2, Ww2, bw2):
    pad = NPAD - N
    x2c = jnp.pad(jnp.sum(point * point, axis=1)[None, :],
                  ((0, 0), (0, pad)))
    a_pad = jnp.pad(-2.0 * point, ((0, pad), (0, 5)))
    bt_pad = jnp.pad(point, ((0, pad), (0, 5))).T
    pt_pad = jnp.pad(point, ((0, pad), (0, 125)))
    feat_pad = jnp.pad(feat, ((0, pad), (0, 0)))

    row = lambda v: v.reshape(1, -1)
    pad16 = lambda v: jnp.pad(v, (0, 16 - v.shape[0])).reshape(1, 16)

    nchunks = 2
    cw = NPAD // nchunks
    gfeats, gpts = [], []
    for h in range(nchunks):
        idx_h = _knn_topk(a_pad[h * cw:(h + 1) * cw], bt_pad, x2c)
        g1, g2 = _sc_gather(feat_pad, pt_pad, idx_h.reshape(-1))
        gfeats.append(g1)
        gpts.append(g2)

    Wp1p = jnp.zeros((16, 16), jnp.float32).at[:3, :3].set(Wp1)
    Wp2p = jnp.zeros((16, COUT), jnp.float32).at[:3, :].set(Wp2)
    bmat = (jax.lax.broadcasted_iota(jnp.int32, (16, COUT), 1) % 16
            == jax.lax.broadcasted_iota(jnp.int32, (16, COUT), 0)
            ).astype(jnp.float32)

    params = [Wq, row(bq), Wk, row(bk), Wv, row(bv),
              Wp1p, pad16(bp1), pad16(gp1), pad16(bep1), Wp2p, row(bp2),
              row(gw1), row(bew1), Ww1, row(bw1),
              pad16(gw2), pad16(bew2), Ww2, pad16(bw2), bmat]
    outs = [_attention(gfeats[h], gpts[h], feat_pad[h * cw:(h + 1) * cw],
                       pt_pad[h * cw:(h + 1) * cw], params)
            for h in range(nchunks)]
    return jnp.concatenate(outs, axis=0)[:N]
